# baseline (speedup 1.0000x reference)
import functools

import jax
import jax.numpy as jnp
from jax import lax
from jax.experimental import pallas as pl
from jax.experimental.pallas import tpu as pltpu
from jax.experimental.pallas import tpu_sc as plsc

D = 128
NC = 2
NS = 16
NW = NC * NS
RING = 5
SP_ROWS = 7936


@functools.lru_cache(maxsize=None)
def _make_kernel(B, V):
    n = B // (NW * 128)
    assert B % (NW * 128) == 0 and n % RING == 0 and n // RING >= 3
    assert V == 8192
    n_outer = n // RING
    hot_rows = V - SP_ROWS

    mesh = plsc.VectorSubcoreMesh(core_axis_name="c", subcore_axis_name="s")

    @functools.partial(
        pl.kernel,
        out_type=jax.ShapeDtypeStruct((B, D), jnp.float32),
        mesh=mesh,
        scratch_types=(
            [pltpu.VMEM((1, 128), jnp.int32) for _ in range(RING)]
            + [pltpu.VMEM((128, D), jnp.float32) for _ in range(RING)]
            + [pltpu.SemaphoreType.DMA for _ in range(3 * RING)]
        ),
    )
    def gather_kernel(ids_hbm, pe_hbm, out_hbm, *scratch):
        idx = scratch[0:RING]
        rows = scratch[RING:2 * RING]
        isem = scratch[2 * RING:3 * RING]
        gsem = scratch[3 * RING:4 * RING]
        osem = scratch[4 * RING:5 * RING]

        sid = lax.axis_index("s")
        wid = sid * NC + lax.axis_index("c")
        row_base = wid * n
        lanes = lax.iota(jnp.int32, 16)

        def fl(g, b):
            pltpu.make_async_copy(
                ids_hbm.at[pl.ds(row_base + g, 1)], idx[b], isem[b]).start()

        def wi(b):
            pltpu.make_async_copy(
                ids_hbm.at[pl.ds(0, 1)], idx[b], isem[b]).wait()

        def fg(b):
            pltpu.make_async_copy(
                pe_hbm.at[idx[b].at[0]], rows[b], gsem[b]).start()

        def wg(b):
            pltpu.make_async_copy(
                pe_hbm.at[pl.ds(0, 128)], rows[b], gsem[b]).wait()

        def fs(g, b):
            pltpu.make_async_copy(
                rows[b],
                out_hbm.at[pl.ds((row_base + g) * 128, 128)],
                osem[b]).start()

        def ws(b):
            pltpu.make_async_copy(
                rows[b], out_hbm.at[pl.ds(0, 128)], osem[b]).wait()

        def prep(b):
            wi(b); fg(b)

        for b in range(RING):
            fl(b, b)
        prep(0)
        prep(1)

        def step(g, b, with_prev=True, with_fl=True, with_ws=True,
                 with_next=True):
            wg(b)
            if with_prev:
                p = (b + RING - 1) % RING
                fs(g - 1, p)
                if with_fl:
                    fl(g + 4, p)
            if with_ws:
                ws((b + 2) % RING)
            if with_next:
                q = (b + 2) % RING
                wi(q); fg(q)

        step(0, 0, with_prev=False, with_ws=False)
        step(1, 1, with_ws=False)
        step(2, 2, with_ws=False)
        step(3, 3)
        step(4, 4)

        def body(i, carry):
            base = i * RING
            for u in range(RING):
                step(base + u, u)
            return carry

        lax.fori_loop(1, n_outer - 1, body, 0)

        g0 = n - RING
        step(g0 + 0, 0)
        step(g0 + 1, 1, with_fl=False)
        step(g0 + 2, 2, with_fl=False)
        step(g0 + 3, 3, with_fl=False, with_next=False)
        step(g0 + 4, 4, with_fl=False, with_next=False)
        fs(n - 1, 4)
        ws(2); ws(3); ws(4)

    return gather_kernel


@jax.jit
def kernel(ids, pe):
    b, h = ids.shape
    B = b * h
    out = _make_kernel(B, pe.shape[0])(ids.reshape(B // 128, 128), pe)
    return out.reshape(b, h, D)
